# baseline (speedup 1.0000x reference)
import jax
import jax.numpy as jnp
from jax.experimental import pallas as pl
from jax.experimental.pallas import tpu as pltpu

_EMBED_DIM = 1024
_MLP_DIM = 1024
_MHA_DIM = 128
_NEURONS = 4096
_HEADS = 16
_F8 = jnp.float8_e4m3fn
_WSCALE = 64.0


def _fused_kernel(x_ref, w1_ref, w2m_ref, w2h_ref,
                  neurons_ref, heads_ref, wch, wcl, wht):
    i = pl.program_id(0)

    @pl.when(i == 0)
    def _():
        w1m = w1_ref[: _MLP_DIM, :].astype(jnp.bfloat16)
        wct = jax.lax.dot_general(
            w1m, w2m_ref[...].astype(jnp.bfloat16),
            (((0,), (1,)), ((), ())),
            preferred_element_type=jnp.float32) * _WSCALE
        hi = wct.astype(_F8)
        wch[...] = hi
        wcl[...] = (wct - hi.astype(jnp.float32)).astype(_F8)
        w1h = w1_ref[_MLP_DIM:, :].astype(jnp.bfloat16)
        wht[...] = jax.lax.dot_general(
            w1h, w2h_ref[...].astype(jnp.bfloat16),
            (((0,), (1,)), ((), ())),
            preferred_element_type=jnp.float32).astype(jnp.bfloat16)

    @pl.when(i > 0)
    def _():
        x = x_ref[...]
        xh = x.astype(_F8)
        xl = (x - xh.astype(jnp.float32)).astype(_F8)
        dn = (((1,), (0,)), ((), ()))
        acc = jax.lax.dot_general(
            xh, wch[...], dn, preferred_element_type=jnp.float32)
        acc += jax.lax.dot_general(
            xh, wcl[...], dn, preferred_element_type=jnp.float32)
        acc += jax.lax.dot_general(
            xl, wch[...], dn, preferred_element_type=jnp.float32)
        neurons_ref[...] = acc * (1.0 / _WSCALE)
        heads_ref[...] = jax.lax.dot_general(
            x.astype(jnp.bfloat16), wht[...], dn,
            preferred_element_type=jnp.float32)


def kernel(x, W1, ln_gamma, ln_beta, W2_mlp, W2_mha):
    del ln_gamma, ln_beta
    n_tok = x.shape[0]
    bt = 512
    n = n_tok // bt
    neurons, heads = pl.pallas_call(
        _fused_kernel,
        grid=(n + 1,),
        in_specs=[
            pl.BlockSpec((bt, _EMBED_DIM),
                         lambda i: (jnp.maximum(i - 1, 0), 0)),
            pl.BlockSpec((_MLP_DIM + _MHA_DIM, _EMBED_DIM), lambda i: (0, 0)),
            pl.BlockSpec((_NEURONS, _MLP_DIM), lambda i: (0, 0)),
            pl.BlockSpec((_HEADS, _MHA_DIM), lambda i: (0, 0)),
        ],
        out_specs=[
            pl.BlockSpec((bt, _NEURONS),
                         lambda i: (jnp.maximum(i - 1, 0), 0)),
            pl.BlockSpec((bt, _HEADS),
                         lambda i: (jnp.maximum(i - 1, 0), 0)),
        ],
        out_shape=[
            jax.ShapeDtypeStruct((n_tok, _NEURONS), jnp.float32),
            jax.ShapeDtypeStruct((n_tok, _HEADS), jnp.float32),
        ],
        scratch_shapes=[
            pltpu.VMEM((_EMBED_DIM, _NEURONS), _F8),
            pltpu.VMEM((_EMBED_DIM, _NEURONS), _F8),
            pltpu.VMEM((_EMBED_DIM, _HEADS), jnp.bfloat16),
        ],
        compiler_params=pltpu.CompilerParams(
            dimension_semantics=("arbitrary",)),
    )(x, W1, W2_mlp, W2_mha)
    return (neurons, heads)


# speedup vs baseline: 1.3780x; 1.3780x over previous
import jax
import jax.numpy as jnp
from jax.experimental import pallas as pl
from jax.experimental.pallas import tpu as pltpu

_EMBED_DIM = 1024
_MLP_DIM = 1024
_MHA_DIM = 128
_NEURONS = 4096
_HEADS = 16


def _fused_kernel(x_ref, w1_ref, w2m_ref, w2h_ref,
                  neurons_ref, heads_ref, wct, wht):
    i = pl.program_id(0)

    @pl.when(i == 0)
    def _():
        w1m = w1_ref[: _MLP_DIM, :].astype(jnp.bfloat16)
        wct[...] = jax.lax.dot_general(
            w1m, w2m_ref[...].astype(jnp.bfloat16),
            (((0,), (1,)), ((), ())),
            preferred_element_type=jnp.float32).astype(jnp.bfloat16)
        w1h = w1_ref[_MLP_DIM:, :].astype(jnp.bfloat16)
        wht[...] = jax.lax.dot_general(
            w1h, w2h_ref[...].astype(jnp.bfloat16),
            (((0,), (1,)), ((), ())),
            preferred_element_type=jnp.float32).astype(jnp.bfloat16)

    @pl.when(i > 0)
    def _():
        x = x_ref[...].astype(jnp.bfloat16)
        neurons_ref[...] = jax.lax.dot_general(
            x, wct[...], (((1,), (0,)), ((), ())),
            preferred_element_type=jnp.float32)
        heads_ref[...] = jax.lax.dot_general(
            x, wht[...], (((1,), (0,)), ((), ())),
            preferred_element_type=jnp.float32)


def kernel(x, W1, ln_gamma, ln_beta, W2_mlp, W2_mha):
    del ln_gamma, ln_beta
    n_tok = x.shape[0]
    bt = 512
    n = n_tok // bt
    neurons, heads = pl.pallas_call(
        _fused_kernel,
        grid=(n + 1,),
        in_specs=[
            pl.BlockSpec((bt, _EMBED_DIM),
                         lambda i: (jnp.maximum(i - 1, 0), 0)),
            pl.BlockSpec((_MLP_DIM + _MHA_DIM, _EMBED_DIM), lambda i: (0, 0)),
            pl.BlockSpec((_NEURONS, _MLP_DIM), lambda i: (0, 0)),
            pl.BlockSpec((_HEADS, _MHA_DIM), lambda i: (0, 0)),
        ],
        out_specs=[
            pl.BlockSpec((bt, _NEURONS),
                         lambda i: (jnp.maximum(i - 1, 0), 0)),
            pl.BlockSpec((bt, _HEADS),
                         lambda i: (jnp.maximum(i - 1, 0), 0)),
        ],
        out_shape=[
            jax.ShapeDtypeStruct((n_tok, _NEURONS), jnp.float32),
            jax.ShapeDtypeStruct((n_tok, _HEADS), jnp.float32),
        ],
        scratch_shapes=[
            pltpu.VMEM((_EMBED_DIM, _NEURONS), jnp.bfloat16),
            pltpu.VMEM((_EMBED_DIM, _HEADS), jnp.bfloat16),
        ],
        compiler_params=pltpu.CompilerParams(
            dimension_semantics=("arbitrary",)),
    )(x, W1, W2_mlp, W2_mha)
    return (neurons, heads)


# speedup vs baseline: 1.3795x; 1.0010x over previous
import jax
import jax.numpy as jnp
from jax.experimental import pallas as pl
from jax.experimental.pallas import tpu as pltpu

_EMBED_DIM = 1024
_MLP_DIM = 1024
_MHA_DIM = 128
_NEURONS = 4096
_HEADS = 16


def _fused_kernel(x_ref, w1_ref, w2m_ref, w2h_ref,
                  neurons_ref, heads_ref, wcat):
    i = pl.program_id(0)

    @pl.when(i == 0)
    def _():
        w1m = w1_ref[: _MLP_DIM, :].astype(jnp.bfloat16)
        wcat[:, :_NEURONS] = jax.lax.dot_general(
            w1m, w2m_ref[...].astype(jnp.bfloat16),
            (((0,), (1,)), ((), ())),
            preferred_element_type=jnp.float32).astype(jnp.bfloat16)
        w1h = w1_ref[_MLP_DIM:, :].astype(jnp.bfloat16)
        wh = jax.lax.dot_general(
            w1h, w2h_ref[...].astype(jnp.bfloat16),
            (((0,), (1,)), ((), ())),
            preferred_element_type=jnp.float32).astype(jnp.bfloat16)
        wcat[:, _NEURONS:] = jnp.pad(wh, ((0, 0), (0, 128 - _HEADS)))

    @pl.when(i > 0)
    def _():
        x = x_ref[...].astype(jnp.bfloat16)
        out = jax.lax.dot_general(
            x, wcat[...], (((1,), (0,)), ((), ())),
            preferred_element_type=jnp.float32)
        neurons_ref[...] = out[:, :_NEURONS]
        heads_ref[...] = out[:, _NEURONS:_NEURONS + _HEADS]


def kernel(x, W1, ln_gamma, ln_beta, W2_mlp, W2_mha):
    del ln_gamma, ln_beta
    n_tok = x.shape[0]
    bt = 512
    n = n_tok // bt
    neurons, heads = pl.pallas_call(
        _fused_kernel,
        grid=(n + 1,),
        in_specs=[
            pl.BlockSpec((bt, _EMBED_DIM),
                         lambda i: (jnp.maximum(i - 1, 0), 0)),
            pl.BlockSpec((_MLP_DIM + _MHA_DIM, _EMBED_DIM), lambda i: (0, 0)),
            pl.BlockSpec((_NEURONS, _MLP_DIM), lambda i: (0, 0)),
            pl.BlockSpec((_HEADS, _MHA_DIM), lambda i: (0, 0)),
        ],
        out_specs=[
            pl.BlockSpec((bt, _NEURONS),
                         lambda i: (jnp.maximum(i - 1, 0), 0)),
            pl.BlockSpec((bt, _HEADS),
                         lambda i: (jnp.maximum(i - 1, 0), 0)),
        ],
        out_shape=[
            jax.ShapeDtypeStruct((n_tok, _NEURONS), jnp.float32),
            jax.ShapeDtypeStruct((n_tok, _HEADS), jnp.float32),
        ],
        scratch_shapes=[
            pltpu.VMEM((_EMBED_DIM, _NEURONS + 128), jnp.bfloat16),
        ],
        compiler_params=pltpu.CompilerParams(
            dimension_semantics=("arbitrary",)),
    )(x, W1, W2_mlp, W2_mha)
    return (neurons, heads)
